# baseline (speedup 1.0000x reference)
import jax
import jax.numpy as jnp
from jax.experimental import pallas as pl
from jax.experimental.pallas import tpu as pltpu

_NUM_EXPERTS = 64
_TM = 1024
_NS = 2


def _top2(logits):
    e = logits.shape[-1]
    iota = jax.lax.broadcasted_iota(jnp.int32, logits.shape, 1)
    m1 = jnp.max(logits, axis=-1, keepdims=True)
    i1 = jnp.min(jnp.where(logits == m1, iota, e), axis=-1, keepdims=True)
    masked = jnp.where(iota == i1, -jnp.inf, logits)
    m2 = jnp.max(masked, axis=-1, keepdims=True)
    i2 = jnp.min(jnp.where(masked == m2, iota, e), axis=-1, keepdims=True)
    z = jnp.sum(jnp.exp(logits - m1), axis=-1, keepdims=True)
    idx = jnp.concatenate([i1, i2], axis=-1)
    score = jnp.concatenate([1.0 / z, jnp.exp(m2 - m1) / z], axis=-1)
    return idx, score


def _router_block(*refs):
    x_refs = refs[:_NS]
    w_ref, b_ref, idx_ref, score_ref = refs[_NS:]
    w = w_ref[...]
    bias = b_ref[...]
    for s in range(_NS):
        logits = jax.lax.dot_general(
            x_refs[s][...], w, (((1,), (1,)), ((), ())),
            preferred_element_type=jnp.float32,
        )
        logits = logits + bias
        idx, score = _top2(logits)
        idx_ref[pl.ds(s * _TM, _TM), :] = idx
        score_ref[pl.ds(s * _TM, _TM), :] = score


@jax.jit
def kernel(x, W, b):
    tokens, d_model = x.shape
    b2 = b.reshape(1, _NUM_EXPERTS)
    grid = (tokens // (_TM * _NS),)
    x_specs = [
        pl.BlockSpec((_TM, d_model), lambda i, s=s: (i * _NS + s, 0))
        for s in range(_NS)
    ]
    idx, scores = pl.pallas_call(
        _router_block,
        grid=grid,
        in_specs=x_specs + [
            pl.BlockSpec((_NUM_EXPERTS, d_model), lambda i: (0, 0)),
            pl.BlockSpec((1, _NUM_EXPERTS), lambda i: (0, 0)),
        ],
        out_specs=[
            pl.BlockSpec((_TM * _NS, 2), lambda i: (i, 0)),
            pl.BlockSpec((_TM * _NS, 2), lambda i: (i, 0)),
        ],
        out_shape=[
            jax.ShapeDtypeStruct((tokens, 2), jnp.int32),
            jax.ShapeDtypeStruct((tokens, 2), jnp.float32),
        ],
        compiler_params=pltpu.CompilerParams(
            dimension_semantics=("arbitrary",),
        ),
    )(*([x] * _NS), W, b2)
    return idx, scores


# speedup vs baseline: 1.0050x; 1.0050x over previous
import jax
import jax.numpy as jnp
from jax.experimental import pallas as pl
from jax.experimental.pallas import tpu as pltpu

_NUM_EXPERTS = 64
_TM = 512
_NS = 4


def _top2(logits):
    e = logits.shape[-1]
    iota = jax.lax.broadcasted_iota(jnp.int32, logits.shape, 1)
    m1 = jnp.max(logits, axis=-1, keepdims=True)
    i1 = jnp.min(jnp.where(logits == m1, iota, e), axis=-1, keepdims=True)
    masked = jnp.where(iota == i1, -jnp.inf, logits)
    m2 = jnp.max(masked, axis=-1, keepdims=True)
    i2 = jnp.min(jnp.where(masked == m2, iota, e), axis=-1, keepdims=True)
    z = jnp.sum(jnp.exp(logits - m1), axis=-1, keepdims=True)
    idx = jnp.concatenate([i1, i2], axis=-1)
    score = jnp.concatenate([1.0 / z, jnp.exp(m2 - m1) / z], axis=-1)
    return idx, score


def _router_block(*refs):
    x_refs = refs[:_NS]
    w_ref, b_ref, idx_ref, score_ref = refs[_NS:]
    w = w_ref[...]
    bias = b_ref[...]
    for s in range(_NS):
        logits = jax.lax.dot_general(
            x_refs[s][...], w, (((1,), (1,)), ((), ())),
            preferred_element_type=jnp.float32,
        )
        logits = logits + bias
        idx, score = _top2(logits)
        idx_ref[pl.ds(s * _TM, _TM), :] = idx
        score_ref[pl.ds(s * _TM, _TM), :] = score


@jax.jit
def kernel(x, W, b):
    tokens, d_model = x.shape
    b2 = b.reshape(1, _NUM_EXPERTS)
    grid = (tokens // (_TM * _NS),)
    x_specs = [
        pl.BlockSpec((_TM, d_model), lambda i, s=s: (i * _NS + s, 0))
        for s in range(_NS)
    ]
    idx, scores = pl.pallas_call(
        _router_block,
        grid=grid,
        in_specs=x_specs + [
            pl.BlockSpec((_NUM_EXPERTS, d_model), lambda i: (0, 0)),
            pl.BlockSpec((1, _NUM_EXPERTS), lambda i: (0, 0)),
        ],
        out_specs=[
            pl.BlockSpec((_TM * _NS, 2), lambda i: (i, 0)),
            pl.BlockSpec((_TM * _NS, 2), lambda i: (i, 0)),
        ],
        out_shape=[
            jax.ShapeDtypeStruct((tokens, 2), jnp.int32),
            jax.ShapeDtypeStruct((tokens, 2), jnp.float32),
        ],
        compiler_params=pltpu.CompilerParams(
            dimension_semantics=("arbitrary",),
        ),
    )(*([x] * _NS), W, b2)
    return idx, scores
